# baseline (speedup 1.0000x reference)
import functools

import jax
import jax.numpy as jnp
from jax import lax
from jax.experimental import pallas as pl
from jax.experimental.pallas import tpu as pltpu
from jax.experimental.pallas import tpu_sc as plsc

B = 16384
H = 512
NT = 5
NL = 6
NC = 2
NS = 16
L = 16
NW = NC * NS
BPW = B // NW
G = 16
SG = 2
GB = G * SG
NB = BPW // GB
HC = H // L


@functools.partial(
    pl.kernel,
    out_type=jax.ShapeDtypeStruct((B, H), jnp.float32),
    mesh=plsc.VectorSubcoreMesh(core_axis_name="c", subcore_axis_name="s"),
    scratch_types=[
        pltpu.VMEM((BPW,), jnp.int32),
        pltpu.VMEM((BPW,), jnp.int32),
        pltpu.VMEM((BPW + L,), jnp.float32),
        pltpu.VMEM((NT, H), jnp.float32),
        pltpu.VMEM((NL, H), jnp.float32),
        pltpu.VMEM((H,), jnp.float32),
        pltpu.VMEM((H,), jnp.float32),
        pltpu.VMEM((NT * NL * H,), jnp.float32),
        pltpu.VMEM((BPW + L,), jnp.int32),
        pltpu.VMEM((GB, H), jnp.float32),
        pltpu.VMEM((GB, H), jnp.float32),
        pltpu.SemaphoreType.DMA,
        pltpu.SemaphoreType.DMA,
    ],
)
def _sc_embed(task_hbm, lang_hbm, comp_hbm, ttab_hbm, ltab_hbm, w_hbm, b_hbm,
              out_hbm,
              ids_v, lids_v, comp_v, ttab_v, ltab_v, w_v, b_v, combo_v,
              offs_v, obuf0_v, obuf1_v, sem0, sem1):
    cid = lax.axis_index("c")
    sid = lax.axis_index("s")
    wid = sid * NC + cid
    base = wid * BPW

    in_copies = (
        pltpu.async_copy(task_hbm.at[pl.ds(base, BPW)], ids_v, sem0),
        pltpu.async_copy(lang_hbm.at[pl.ds(base, BPW)], lids_v, sem0),
        pltpu.async_copy(
            comp_hbm.at[pl.ds(base, BPW)], comp_v.at[pl.ds(0, BPW)], sem0),
        pltpu.async_copy(ttab_hbm, ttab_v, sem0),
        pltpu.async_copy(ltab_hbm, ltab_v, sem0),
        pltpu.async_copy(w_hbm, w_v, sem0),
        pltpu.async_copy(b_hbm, b_v, sem0),
    )
    for c in in_copies:
        c.wait()

    @plsc.parallel_loop(0, NT * NL)
    def _combo_row(row):
        t = row // NL
        l = row - t * NL
        ro = row * H
        for hcv in range(HC):
            hh = hcv * L
            combo_v[pl.ds(ro + hh, L)] = (
                ttab_v[t, pl.ds(hh, L)]
                + ltab_v[l, pl.ds(hh, L)]
                + b_v[pl.ds(hh, L)]
            )

    @plsc.parallel_loop(0, BPW // L)
    def _offsets(g):
        rbase = g * L
        tvec = ids_v[pl.ds(rbase, L)]
        lvec = lids_v[pl.ds(rbase, L)]
        offs_v[pl.ds(rbase, L)] = (tvec * NL + lvec) * H

    def block_pair(bp, carry):
        for buf, sem, half in ((obuf0_v, sem0, 0), (obuf1_v, sem1, 1)):
            blk = bp * 2 + half
            bbase = blk * GB

            @pl.when(bp > 0)
            def _wait():
                pltpu.make_async_copy(
                    buf, out_hbm.at[pl.ds(0, GB), :], sem).wait()

            for q in range(SG):
                rbase = bbase + q * G
                offs = [offs_v[pl.ds(rbase + r, L)][0] for r in range(G)]
                ccs = [comp_v[pl.ds(rbase + r, L)][0] for r in range(G)]

                @plsc.parallel_loop(0, HC, unroll=1)
                def _chunks(hcv, _q=q, _offs=offs, _ccs=ccs):
                    hh = hcv * L
                    wv = w_v[pl.ds(hh, L)]
                    for r in range(G):
                        buf[_q * G + r, pl.ds(hh, L)] = (
                            combo_v[pl.ds(_offs[r] + hh, L)] + _ccs[r] * wv
                        )

            pltpu.async_copy(
                buf, out_hbm.at[pl.ds(base + bbase, GB), :], sem)
        return carry

    lax.fori_loop(0, NB // 2, block_pair, 0)
    pltpu.make_async_copy(obuf0_v, out_hbm.at[pl.ds(0, GB), :], sem0).wait()
    pltpu.make_async_copy(obuf1_v, out_hbm.at[pl.ds(0, GB), :], sem1).wait()


def kernel(task_ids, language_ids, complexity, task_table, lang_table, W, b):
    return _sc_embed(
        task_ids.astype(jnp.int32),
        language_ids.astype(jnp.int32),
        complexity.reshape(B),
        task_table,
        lang_table,
        W.reshape(H),
        b,
    )


# speedup vs baseline: 1.0432x; 1.0432x over previous
import functools

import jax
import jax.numpy as jnp
from jax import lax
from jax.experimental import pallas as pl
from jax.experimental.pallas import tpu as pltpu
from jax.experimental.pallas import tpu_sc as plsc

B = 16384
H = 512
NT = 5
NL = 6
NC = 2
NS = 16
L = 16
NW = NC * NS
BPW = B // NW
G = 16
SG = 1
GB = G * SG
NB = BPW // GB
HC = H // L


@functools.partial(
    pl.kernel,
    out_type=jax.ShapeDtypeStruct((B, H), jnp.float32),
    mesh=plsc.VectorSubcoreMesh(core_axis_name="c", subcore_axis_name="s"),
    scratch_types=[
        pltpu.VMEM((BPW,), jnp.int32),
        pltpu.VMEM((BPW,), jnp.int32),
        pltpu.VMEM((BPW + L,), jnp.float32),
        pltpu.VMEM((NT, H), jnp.float32),
        pltpu.VMEM((NL, H), jnp.float32),
        pltpu.VMEM((H,), jnp.float32),
        pltpu.VMEM((H,), jnp.float32),
        pltpu.VMEM((NT * NL * H,), jnp.float32),
        pltpu.VMEM((BPW + L,), jnp.int32),
        pltpu.VMEM((GB, H), jnp.float32),
        pltpu.VMEM((GB, H), jnp.float32),
        pltpu.SemaphoreType.DMA,
        pltpu.SemaphoreType.DMA,
    ],
)
def _sc_embed(task_hbm, lang_hbm, comp_hbm, ttab_hbm, ltab_hbm, w_hbm, b_hbm,
              out_hbm,
              ids_v, lids_v, comp_v, ttab_v, ltab_v, w_v, b_v, combo_v,
              offs_v, obuf0_v, obuf1_v, sem0, sem1):
    cid = lax.axis_index("c")
    sid = lax.axis_index("s")
    wid = sid * NC + cid
    base = wid * BPW

    in_copies = (
        pltpu.async_copy(task_hbm.at[pl.ds(base, BPW)], ids_v, sem0),
        pltpu.async_copy(lang_hbm.at[pl.ds(base, BPW)], lids_v, sem0),
        pltpu.async_copy(
            comp_hbm.at[pl.ds(base, BPW)], comp_v.at[pl.ds(0, BPW)], sem0),
        pltpu.async_copy(ttab_hbm, ttab_v, sem0),
        pltpu.async_copy(ltab_hbm, ltab_v, sem0),
        pltpu.async_copy(w_hbm, w_v, sem0),
        pltpu.async_copy(b_hbm, b_v, sem0),
    )
    for c in in_copies:
        c.wait()

    @plsc.parallel_loop(0, NT * NL)
    def _combo_row(row):
        t = row // NL
        l = row - t * NL
        ro = row * H
        for hcv in range(HC):
            hh = hcv * L
            combo_v[pl.ds(ro + hh, L)] = (
                ttab_v[t, pl.ds(hh, L)]
                + ltab_v[l, pl.ds(hh, L)]
                + b_v[pl.ds(hh, L)]
            )

    @plsc.parallel_loop(0, BPW // L)
    def _offsets(g):
        rbase = g * L
        tvec = ids_v[pl.ds(rbase, L)]
        lvec = lids_v[pl.ds(rbase, L)]
        offs_v[pl.ds(rbase, L)] = (tvec * NL + lvec) * H

    def block_pair(bp, carry):
        for buf, sem, half in ((obuf0_v, sem0, 0), (obuf1_v, sem1, 1)):
            blk = bp * 2 + half
            bbase = blk * GB

            @pl.when(bp > 0)
            def _wait():
                pltpu.make_async_copy(
                    buf, out_hbm.at[pl.ds(0, GB), :], sem).wait()

            for q in range(SG):
                rbase = bbase + q * G
                offs = [offs_v[pl.ds(rbase + r, L)][0] for r in range(G)]
                ccs = [comp_v[pl.ds(rbase + r, L)][0] for r in range(G)]

                @plsc.parallel_loop(0, HC, unroll=1)
                def _chunks(hcv, _q=q, _offs=offs, _ccs=ccs):
                    hh = hcv * L
                    wv = w_v[pl.ds(hh, L)]
                    for r in range(G):
                        buf[_q * G + r, pl.ds(hh, L)] = (
                            combo_v[pl.ds(_offs[r] + hh, L)] + _ccs[r] * wv
                        )

            pltpu.async_copy(
                buf, out_hbm.at[pl.ds(base + bbase, GB), :], sem)
        return carry

    lax.fori_loop(0, NB // 2, block_pair, 0)
    pltpu.make_async_copy(obuf0_v, out_hbm.at[pl.ds(0, GB), :], sem0).wait()
    pltpu.make_async_copy(obuf1_v, out_hbm.at[pl.ds(0, GB), :], sem1).wait()


def kernel(task_ids, language_ids, complexity, task_table, lang_table, W, b):
    return _sc_embed(
        task_ids.astype(jnp.int32),
        language_ids.astype(jnp.int32),
        complexity.reshape(B),
        task_table,
        lang_table,
        W.reshape(H),
        b,
    )
